# Initial kernel scaffold; baseline (speedup 1.0000x reference)
import jax
import jax.numpy as jnp
from jax.experimental import pallas as pl


def kernel(audio_input, hidden_state, cell_state, stft_w, enc_w1, enc_b1, enc_w2, enc_b2, enc_w3, enc_b3, enc_w4, enc_b4, w_ih, w_hh, b_ih, b_hh, out_w, out_b):
    raise NotImplementedError("write your pallas kernel here")


# speedup vs baseline: 3.6912x; 3.6912x over previous
import jax
import jax.numpy as jnp
from jax.experimental import pallas as pl
from jax.experimental.pallas import tpu as pltpu

_BB = 256
_NB = 129


def _vad_block(audio_ref, h0_ref, c0_ref, wstft_ref, w0stft_ref,
               bigw1_ref, b1_ref, w2_ref, b2_ref, w3_ref, b3_ref,
               w4_ref, b4_ref, wl_ref, bl_ref, ow_ref, ob_ref,
               fin_ref, hout_ref, cout_ref):
    audio = audio_ref[...]
    h = h0_ref[...]
    c = c0_ref[...]
    bb = audio.shape[0]
    acc = jnp.ones((bb, 1), jnp.float32)
    wt = wstft_ref[...]
    w0 = w0stft_ref[...]
    bigw1 = bigw1_ref[...]
    w2 = w2_ref[...]
    w3 = w3_ref[...]
    w4 = w4_ref[...]
    wl = wl_ref[...]

    for i in range(8):
        base = 512 * i
        f0 = audio[:, base:base + 256]
        f1 = audio[:, base + 64:base + 320]
        f2 = audio[:, base + 192:base + 448]
        f3 = audio[:, base + 320:base + 576]
        s0 = jnp.dot(f0, w0, preferred_element_type=jnp.float32)
        s1 = jnp.dot(f1, wt, preferred_element_type=jnp.float32)
        s2 = jnp.dot(f2, wt, preferred_element_type=jnp.float32)
        s3 = jnp.dot(f3, wt, preferred_element_type=jnp.float32)
        mags = [jnp.sqrt(s[:, :256] ** 2 + s[:, 256:] ** 2)
                for s in (s0, s1, s2, s3)]
        mcat = jnp.concatenate(mags, axis=1)
        h1 = jnp.maximum(
            jnp.dot(mcat, bigw1, preferred_element_type=jnp.float32)
            + b1_ref[...], 0.0)
        h2 = jnp.maximum(
            jnp.dot(h1, w2, preferred_element_type=jnp.float32)
            + b2_ref[...], 0.0)
        h3 = jnp.maximum(
            jnp.dot(h2, w3, preferred_element_type=jnp.float32)
            + b3_ref[...], 0.0)
        h4 = jnp.maximum(
            jnp.dot(h3, w4, preferred_element_type=jnp.float32)
            + b4_ref[...], 0.0)
        xin = jnp.concatenate([h4, h], axis=1)
        gates = jnp.dot(xin, wl, preferred_element_type=jnp.float32) \
            + bl_ref[...]
        i_g = jax.nn.sigmoid(gates[:, 0:128])
        f_g = jax.nn.sigmoid(gates[:, 128:256])
        g_g = jnp.tanh(gates[:, 256:384])
        o_g = jax.nn.sigmoid(gates[:, 384:512])
        c = f_g * c + i_g * g_g
        h = o_g * jnp.tanh(c)
        p = jax.nn.sigmoid(
            jnp.sum(h * ow_ref[...], axis=1, keepdims=True) + ob_ref[...])
        acc = acc * (1.0 - p)

    fin_ref[...] = 1.0 - acc
    hout_ref[...] = h
    cout_ref[...] = c


def kernel(audio_input, hidden_state, cell_state, stft_w,
           enc_w1, enc_b1, enc_w2, enc_b2, enc_w3, enc_b3, enc_w4, enc_b4,
           w_ih, w_hh, b_ih, b_hh, out_w, out_b):
    f32 = jnp.float32
    b = audio_input.shape[0]

    wt = stft_w[:, 0, :].T
    c0 = wt[64:256].at[1:65].add(jnp.flip(wt[0:64], axis=0))
    c0p = jnp.concatenate([c0, jnp.zeros((64, 258), f32)], axis=0)

    def bank(w):
        out = jnp.zeros((256, 512), f32)
        out = out.at[:, 0:129].set(w[:, 0:129])
        out = out.at[:, 256:256 + _NB].set(w[:, 129:258])
        return out

    wstft = bank(wt)
    w0stft = bank(c0p)

    w1d = jnp.transpose(enc_w1, (2, 1, 0))
    w1p = jnp.zeros((3, 256, 128), f32).at[:, 0:129, :].set(w1d)
    blocks = []
    for s in range(4):
        row = []
        for t in range(4):
            d = s - t + 1
            if 0 <= d <= 2:
                row.append(w1p[d])
            else:
                row.append(jnp.zeros((256, 128), f32))
        blocks.append(jnp.concatenate(row, axis=1))
    bigw1 = jnp.concatenate(blocks, axis=0)
    b1r = jnp.tile(enc_b1, 4)[None]

    w2d = jnp.transpose(enc_w2, (2, 1, 0))
    z128 = jnp.zeros((128, 64), f32)
    w2big = jnp.concatenate([
        jnp.concatenate([w2d[1], z128], axis=1),
        jnp.concatenate([w2d[2], w2d[0]], axis=1),
        jnp.concatenate([z128, w2d[1]], axis=1),
        jnp.concatenate([z128, w2d[2]], axis=1),
    ], axis=0)
    b2b = jnp.concatenate([enc_b2, enc_b2])[None]

    w3d = jnp.transpose(enc_w3, (2, 1, 0))
    w3big = jnp.concatenate([w3d[1], w3d[2]], axis=0)
    b3r = enc_b3[None]

    w4p = jnp.transpose(enc_w4, (2, 1, 0))[1]
    b4r = enc_b4[None]

    wl = jnp.concatenate([w_ih.T, w_hh.T], axis=0)
    blr = (b_ih + b_hh)[None]
    owr = out_w
    obr = out_b[None]

    grid = (b // _BB,)

    def bcast(shape):
        nd = len(shape)
        return pl.BlockSpec(shape, lambda i: (0,) * nd)

    fin, h_fin, c_fin = pl.pallas_call(
        _vad_block,
        grid=grid,
        in_specs=[
            pl.BlockSpec((_BB, 4160), lambda i: (i, 0)),
            pl.BlockSpec((_BB, 128), lambda i: (i, 0)),
            pl.BlockSpec((_BB, 128), lambda i: (i, 0)),
            bcast((256, 512)),
            bcast((256, 512)),
            bcast((1024, 512)),
            bcast((1, 512)),
            bcast((512, 128)),
            bcast((1, 128)),
            bcast((128, 64)),
            bcast((1, 64)),
            bcast((64, 128)),
            bcast((1, 128)),
            bcast((256, 512)),
            bcast((1, 512)),
            bcast((1, 128)),
            bcast((1, 1)),
        ],
        out_specs=[
            pl.BlockSpec((_BB, 1), lambda i: (i, 0)),
            pl.BlockSpec((_BB, 128), lambda i: (i, 0)),
            pl.BlockSpec((_BB, 128), lambda i: (i, 0)),
        ],
        out_shape=[
            jax.ShapeDtypeStruct((b, 1), f32),
            jax.ShapeDtypeStruct((b, 128), f32),
            jax.ShapeDtypeStruct((b, 128), f32),
        ],
        compiler_params=pltpu.CompilerParams(
            dimension_semantics=("parallel",),
            vmem_limit_bytes=48 * 1024 * 1024,
        ),
        name="vad256ms_fused",
    )(audio_input, hidden_state, cell_state, wstft, w0stft,
      bigw1, b1r, w2big, b2b, w3big, b3r, w4p, b4r, wl, blr, owr, obr)

    return fin[:, :, None], h_fin, c_fin
